# baseline (speedup 1.0000x reference)
import functools

import jax
import jax.numpy as jnp
from jax import lax
from jax.experimental import pallas as pl
from jax.experimental.pallas import tpu as pltpu

_K = 32

_F32 = jnp.float32
_I32 = jnp.int32


def _prep_kernel(c_ref, w_ref, bmat_ref, l2_ref, nm_ref):
    c = c_ref[...]
    hi = jax.lax.Precision.HIGHEST
    g = lax.dot_general(c, c, (((0,), (0,)), ((), ())),
                        preferred_element_type=_F32, precision=hi)
    n_c = g.shape[0]
    absg = jnp.abs(g)
    norm_inf = jnp.max(jnp.sum(absg, axis=1))
    norm_one = jnp.max(jnp.sum(absg, axis=0))
    x0 = g * (1.0 / (norm_inf * norm_one))

    def body(_, x):
        xg = jnp.dot(x, g, preferred_element_type=_F32, precision=hi)
        return 2.0 * x - jnp.dot(xg, x, preferred_element_type=_F32,
                                 precision=hi)

    ginv = lax.fori_loop(0, 24, body, x0)
    ctw = lax.dot_general(c, w_ref[...], (((0,), (0,)), ((), ())),
                          preferred_element_type=_F32, precision=hi)
    bmat_ref[...] = jnp.dot(ginv, ctw, preferred_element_type=_F32,
                            precision=hi)

    rows = lax.broadcasted_iota(_I32, g.shape, 0)
    cols = lax.broadcasted_iota(_I32, g.shape, 1)
    diag = rows == cols
    tr = jnp.sum(jnp.where(diag, g, 0.0))
    total = jnp.sum(g)
    denom = 1.0 / float(n_c * n_c)
    l2_ref[...] = jnp.reshape((total - tr) * denom, (1, 1))
    nm_ref[...] = jnp.reshape(tr * denom, (1, 1))


def _fused_kernel(e_ref, c_ref, w_ref, bmat_ref, b_ref, x_ref,
                  orig_ref, y_ref, l1_ref, s_scr, cr_scr, st_scr,
                  *, n_total, k, xtile, fill_steps, n_bisect):
    i = pl.program_id(0)
    n_steps = pl.num_programs(0)
    n_c = c_ref.shape[1]
    kf = float(k)
    inf = jnp.float32(jnp.inf)
    n_pad = fill_steps * xtile
    chunk = 6400
    starts = list(range(0, n_pad, chunk))

    e = e_ref[...]
    b = b_ref[...]
    orig_ref[...] = jnp.dot(e, w_ref[...], preferred_element_type=_F32) + b
    a = jnp.dot(e, c_ref[...], preferred_element_type=_F32)
    y_ref[...] = jnp.dot(a, bmat_ref[...], preferred_element_type=_F32) + b

    @pl.when(i < fill_steps)
    def _fill():
        x = x_ref[...]
        cr = lax.dot_general(c_ref[...], x, (((0,), (0,)), ((), ())),
                             preferred_element_type=_F32)
        xsq = jnp.sum(x * x, axis=0, keepdims=True)
        s = xsq - 2.0 * cr
        col = lax.broadcasted_iota(_I32, s.shape, 1) + i * xtile
        valid = col < n_total
        s_pad = jnp.where(valid, s, inf)
        s_scr[:, pl.ds(i * xtile, xtile)] = s_pad
        cr_scr[:, pl.ds(i * xtile, xtile)] = cr

        mn_c = jnp.min(s_pad, axis=1, keepdims=True)
        mx_c = jnp.max(jnp.where(valid, s, -inf), axis=1, keepdims=True)
        first = i == 0
        st_scr[:, 0:1] = jnp.where(first, mn_c,
                                   jnp.minimum(st_scr[:, 0:1], mn_c))
        st_scr[:, 1:2] = jnp.where(first, mx_c,
                                   jnp.maximum(st_scr[:, 1:2], mx_c))

    bis_steps = n_steps - 1 - fill_steps
    per_step = -(-n_bisect // max(bis_steps, 1))

    @pl.when(jnp.logical_and(i >= fill_steps, i < n_steps - 1))
    def _bisect():
        first = i == fill_steps
        lo = jnp.where(first, st_scr[:, 0:1] - 1.0, st_scr[:, 2:3])
        hi = jnp.where(first, st_scr[:, 1:2], st_scr[:, 3:4])
        for _ in range(per_step):
            mid = 0.5 * lo + 0.5 * hi
            cnt = jnp.zeros((n_c, 1), dtype=_F32)
            for st in starts:
                sc = s_scr[:, pl.ds(st, chunk)]
                cnt = cnt + jnp.sum(jnp.where(sc <= mid, 1.0, 0.0),
                                    axis=1, keepdims=True)
            pred = cnt >= kf
            lo = jnp.where(pred, lo, mid)
            hi = jnp.where(pred, mid, hi)
        st_scr[:, 2:3] = lo
        st_scr[:, 3:4] = hi

    @pl.when(i == n_steps - 1)
    def _final():
        lo = st_scr[:, 2:3]
        hi = st_scr[:, 3:4]
        n_lt = jnp.zeros((n_c, 1), dtype=_F32)
        sum_lt = jnp.zeros((n_c, 1), dtype=_F32)
        n_md = jnp.zeros((n_c, 1), dtype=_F32)
        sum_md = jnp.zeros((n_c, 1), dtype=_F32)
        for st in starts:
            sc = s_scr[:, pl.ds(st, chunk)]
            cc = cr_scr[:, pl.ds(st, chunk)]
            lt = sc <= lo
            md = jnp.logical_and(sc > lo, sc <= hi)
            n_lt = n_lt + jnp.sum(jnp.where(lt, 1.0, 0.0), axis=1,
                                  keepdims=True)
            sum_lt = sum_lt + jnp.sum(jnp.where(lt, cc, 0.0), axis=1,
                                      keepdims=True)
            n_md = n_md + jnp.sum(jnp.where(md, 1.0, 0.0), axis=1,
                                  keepdims=True)
            sum_md = sum_md + jnp.sum(jnp.where(md, cc, 0.0), axis=1,
                                      keepdims=True)
        m = kf - n_lt
        row_total = sum_lt + m * sum_md / jnp.maximum(n_md, 1.0)
        l1_ref[...] = jnp.reshape(jnp.sum(row_total) / (kf * float(n_c)),
                                  (1, 1))


def kernel(train_embedding, concept, train_embeddings_T, W_h, b_h, topk):
    bs, d = train_embedding.shape
    n_c = concept.shape[1]
    n = train_embeddings_T.shape[1]
    n_cls = W_h.shape[1]

    bmat, l2, nm = pl.pallas_call(
        _prep_kernel,
        out_shape=(
            jax.ShapeDtypeStruct((n_c, n_cls), _F32),
            jax.ShapeDtypeStruct((1, 1), _F32),
            jax.ShapeDtypeStruct((1, 1), _F32),
        ),
    )(concept, W_h)

    tile = 512
    grid_m = bs // tile
    xtile = 2560
    fill_steps = (n + xtile - 1) // xtile
    n_pad = fill_steps * xtile
    fs = fill_steps

    orig_pred, y_pred, l1 = pl.pallas_call(
        functools.partial(_fused_kernel, n_total=n, k=_K, xtile=xtile,
                          fill_steps=fill_steps, n_bisect=25),
        grid=(grid_m,),
        in_specs=[
            pl.BlockSpec((tile, d), lambda i: (i, 0)),
            pl.BlockSpec((d, n_c), lambda i: (0, 0)),
            pl.BlockSpec((d, n_cls), lambda i: (0, 0)),
            pl.BlockSpec((n_c, n_cls), lambda i: (0, 0)),
            pl.BlockSpec((1, n_cls), lambda i: (0, 0)),
            pl.BlockSpec((d, xtile),
                         lambda i: (0, jnp.minimum(i, fs - 1))),
        ],
        out_specs=(
            pl.BlockSpec((tile, n_cls), lambda i: (i, 0)),
            pl.BlockSpec((tile, n_cls), lambda i: (i, 0)),
            pl.BlockSpec((1, 1), lambda i: (0, 0)),
        ),
        out_shape=(
            jax.ShapeDtypeStruct((bs, n_cls), _F32),
            jax.ShapeDtypeStruct((bs, n_cls), _F32),
            jax.ShapeDtypeStruct((1, 1), _F32),
        ),
        scratch_shapes=[
            pltpu.VMEM((n_c, n_pad), _F32),
            pltpu.VMEM((n_c, n_pad), _F32),
            pltpu.VMEM((n_c, 128), _F32),
        ],
        compiler_params=pltpu.CompilerParams(
            dimension_semantics=("arbitrary",)),
    )(train_embedding, concept, W_h, bmat, b_h.reshape(1, n_cls),
      train_embeddings_T)

    return (orig_pred, y_pred, l1[0, 0], l2[0, 0], nm[0, 0])


# speedup vs baseline: 1.0186x; 1.0186x over previous
import functools

import jax
import jax.numpy as jnp
from jax import lax
from jax.experimental import pallas as pl
from jax.experimental.pallas import tpu as pltpu

_K = 32

_F32 = jnp.float32
_I32 = jnp.int32


def _prep_kernel(c_ref, w_ref, bmat_ref, l2_ref, nm_ref):
    c = c_ref[...]
    hi = jax.lax.Precision.HIGHEST
    g = lax.dot_general(c, c, (((0,), (0,)), ((), ())),
                        preferred_element_type=_F32, precision=hi)
    n_c = g.shape[0]
    absg = jnp.abs(g)
    norm_inf = jnp.max(jnp.sum(absg, axis=1))
    norm_one = jnp.max(jnp.sum(absg, axis=0))
    x0 = g * (1.0 / (norm_inf * norm_one))

    def body(_, x):
        xg = jnp.dot(x, g, preferred_element_type=_F32, precision=hi)
        return 2.0 * x - jnp.dot(xg, x, preferred_element_type=_F32,
                                 precision=hi)

    ginv = lax.fori_loop(0, 24, body, x0)
    ctw = lax.dot_general(c, w_ref[...], (((0,), (0,)), ((), ())),
                          preferred_element_type=_F32, precision=hi)
    bmat_ref[...] = jnp.dot(ginv, ctw, preferred_element_type=_F32,
                            precision=hi)

    rows = lax.broadcasted_iota(_I32, g.shape, 0)
    cols = lax.broadcasted_iota(_I32, g.shape, 1)
    diag = rows == cols
    tr = jnp.sum(jnp.where(diag, g, 0.0))
    total = jnp.sum(g)
    denom = 1.0 / float(n_c * n_c)
    l2_ref[...] = jnp.reshape((total - tr) * denom, (1, 1))
    nm_ref[...] = jnp.reshape(tr * denom, (1, 1))


def _fused_kernel(e_ref, c_ref, w_ref, bmat_ref, b_ref, x_hbm,
                  orig_ref, y_ref, l1_ref, s_scr, cr_scr, st_scr,
                  x_buf, x_last, sem, *, n_total, k, xtile, fill_steps,
                  n_bisect):
    i = pl.program_id(0)
    n_steps = pl.num_programs(0)
    n_c = c_ref.shape[1]
    kf = float(k)
    last_w = n_total - (fill_steps - 1) * xtile
    chunk = 6400
    starts = [(st, min(chunk, n_total - st))
              for st in range(0, n_total, chunk)]

    def x_copy(ci, buf, wait):
        @pl.when(ci < fill_steps - 1)
        def _full():
            cp = pltpu.make_async_copy(
                x_hbm.at[:, pl.ds(ci * xtile, xtile)], x_buf.at[buf],
                sem.at[buf])
            cp.wait() if wait else cp.start()

        @pl.when(ci == fill_steps - 1)
        def _part():
            cp = pltpu.make_async_copy(
                x_hbm.at[:, pl.ds((fill_steps - 1) * xtile, last_w)],
                x_last, sem.at[2])
            cp.wait() if wait else cp.start()

    @pl.when(i == 0)
    def _prime():
        x_copy(jnp.int32(0), jnp.int32(0), False)

    @pl.when(i < fill_steps - 1)
    def _next():
        x_copy(i + 1, (i + 1) % 2, False)

    e = e_ref[...]
    b = b_ref[...]
    orig_ref[...] = jnp.dot(e, w_ref[...], preferred_element_type=_F32) + b
    a = jnp.dot(e, c_ref[...], preferred_element_type=_F32)
    y_ref[...] = jnp.dot(a, bmat_ref[...], preferred_element_type=_F32) + b

    def _process(x, start):
        cr = lax.dot_general(c_ref[...], x, (((0,), (0,)), ((), ())),
                             preferred_element_type=_F32)
        xsq = jnp.sum(x * x, axis=0, keepdims=True)
        s = xsq - 2.0 * cr
        w = x.shape[1]
        s_scr[:, pl.ds(start, w)] = s
        cr_scr[:, pl.ds(start, w)] = cr
        mn_c = jnp.min(s, axis=1, keepdims=True)
        mx_c = jnp.max(s, axis=1, keepdims=True)
        first = i == 0
        st_scr[:, 0:1] = jnp.where(first, mn_c,
                                   jnp.minimum(st_scr[:, 0:1], mn_c))
        st_scr[:, 1:2] = jnp.where(first, mx_c,
                                   jnp.maximum(st_scr[:, 1:2], mx_c))

    @pl.when(i < fill_steps - 1)
    def _fill():
        x_copy(i, i % 2, True)
        _process(x_buf[i % 2], i * xtile)

    @pl.when(i == fill_steps - 1)
    def _fill_last():
        x_copy(i, jnp.int32(0), True)
        _process(x_last[...], (fill_steps - 1) * xtile)

    bis_steps = n_steps - 1 - fill_steps
    per_step = -(-n_bisect // max(bis_steps, 1))

    @pl.when(jnp.logical_and(i >= fill_steps, i < n_steps - 1))
    def _bisect():
        first = i == fill_steps
        lo = jnp.where(first, st_scr[:, 0:1] - 1.0, st_scr[:, 2:3])
        hi = jnp.where(first, st_scr[:, 1:2], st_scr[:, 3:4])
        for _ in range(per_step):
            mid = 0.5 * lo + 0.5 * hi
            cnt = jnp.zeros((n_c, 1), dtype=_F32)
            for st, w in starts:
                sc = s_scr[:, pl.ds(st, w)]
                cnt = cnt + jnp.sum(jnp.where(sc <= mid, 1.0, 0.0),
                                    axis=1, keepdims=True)
            pred = cnt >= kf
            lo = jnp.where(pred, lo, mid)
            hi = jnp.where(pred, mid, hi)
        st_scr[:, 2:3] = lo
        st_scr[:, 3:4] = hi

    @pl.when(i == n_steps - 1)
    def _final():
        lo = st_scr[:, 2:3]
        hi = st_scr[:, 3:4]
        n_lt = jnp.zeros((n_c, 1), dtype=_F32)
        sum_lt = jnp.zeros((n_c, 1), dtype=_F32)
        n_md = jnp.zeros((n_c, 1), dtype=_F32)
        sum_md = jnp.zeros((n_c, 1), dtype=_F32)
        for st, w in starts:
            sc = s_scr[:, pl.ds(st, w)]
            cc = cr_scr[:, pl.ds(st, w)]
            lt = sc <= lo
            md = jnp.logical_and(sc > lo, sc <= hi)
            n_lt = n_lt + jnp.sum(jnp.where(lt, 1.0, 0.0), axis=1,
                                  keepdims=True)
            sum_lt = sum_lt + jnp.sum(jnp.where(lt, cc, 0.0), axis=1,
                                      keepdims=True)
            n_md = n_md + jnp.sum(jnp.where(md, 1.0, 0.0), axis=1,
                                  keepdims=True)
            sum_md = sum_md + jnp.sum(jnp.where(md, cc, 0.0), axis=1,
                                      keepdims=True)
        m = kf - n_lt
        row_total = sum_lt + m * sum_md / jnp.maximum(n_md, 1.0)
        l1_ref[...] = jnp.reshape(jnp.sum(row_total) / (kf * float(n_c)),
                                  (1, 1))


def kernel(train_embedding, concept, train_embeddings_T, W_h, b_h, topk):
    bs, d = train_embedding.shape
    n_c = concept.shape[1]
    n = train_embeddings_T.shape[1]
    n_cls = W_h.shape[1]

    bmat, l2, nm = pl.pallas_call(
        _prep_kernel,
        out_shape=(
            jax.ShapeDtypeStruct((n_c, n_cls), _F32),
            jax.ShapeDtypeStruct((1, 1), _F32),
            jax.ShapeDtypeStruct((1, 1), _F32),
        ),
    )(concept, W_h)

    tile = 512
    grid_m = bs // tile
    xtile = 2560
    fill_steps = (n + xtile - 1) // xtile

    orig_pred, y_pred, l1 = pl.pallas_call(
        functools.partial(_fused_kernel, n_total=n, k=_K, xtile=xtile,
                          fill_steps=fill_steps, n_bisect=25),
        grid=(grid_m,),
        in_specs=[
            pl.BlockSpec((tile, d), lambda i: (i, 0)),
            pl.BlockSpec((d, n_c), lambda i: (0, 0)),
            pl.BlockSpec((d, n_cls), lambda i: (0, 0)),
            pl.BlockSpec((n_c, n_cls), lambda i: (0, 0)),
            pl.BlockSpec((1, n_cls), lambda i: (0, 0)),
            pl.BlockSpec(memory_space=pl.ANY),
        ],
        out_specs=(
            pl.BlockSpec((tile, n_cls), lambda i: (i, 0)),
            pl.BlockSpec((tile, n_cls), lambda i: (i, 0)),
            pl.BlockSpec((1, 1), lambda i: (0, 0)),
        ),
        out_shape=(
            jax.ShapeDtypeStruct((bs, n_cls), _F32),
            jax.ShapeDtypeStruct((bs, n_cls), _F32),
            jax.ShapeDtypeStruct((1, 1), _F32),
        ),
        scratch_shapes=[
            pltpu.VMEM((n_c, n), _F32),
            pltpu.VMEM((n_c, n), _F32),
            pltpu.VMEM((n_c, 128), _F32),
            pltpu.VMEM((2, d, xtile), _F32),
            pltpu.VMEM((d, n - (fill_steps - 1) * xtile), _F32),
            pltpu.SemaphoreType.DMA((3,)),
        ],
        compiler_params=pltpu.CompilerParams(
            dimension_semantics=("arbitrary",)),
    )(train_embedding, concept, W_h, bmat, b_h.reshape(1, n_cls),
      train_embeddings_T)

    return (orig_pred, y_pred, l1[0, 0], l2[0, 0], nm[0, 0])


# speedup vs baseline: 1.0421x; 1.0231x over previous
import functools

import jax
import jax.numpy as jnp
from jax import lax
from jax.experimental import pallas as pl
from jax.experimental.pallas import tpu as pltpu

_K = 32

_F32 = jnp.float32
_I32 = jnp.int32


def _prep_kernel(c_ref, w_ref, bmat_ref, l2_ref, nm_ref):
    c = c_ref[...]
    hi = jax.lax.Precision.HIGHEST
    g = lax.dot_general(c, c, (((0,), (0,)), ((), ())),
                        preferred_element_type=_F32, precision=hi)
    n_c = g.shape[0]
    absg = jnp.abs(g)
    norm_inf = jnp.max(jnp.sum(absg, axis=1))
    norm_one = jnp.max(jnp.sum(absg, axis=0))
    x0 = g * (1.0 / (norm_inf * norm_one))

    def body(_, x):
        xg = jnp.dot(x, g, preferred_element_type=_F32, precision=hi)
        return 2.0 * x - jnp.dot(xg, x, preferred_element_type=_F32,
                                 precision=hi)

    ginv = lax.fori_loop(0, 24, body, x0)
    ctw = lax.dot_general(c, w_ref[...], (((0,), (0,)), ((), ())),
                          preferred_element_type=_F32, precision=hi)
    bmat_ref[...] = jnp.dot(ginv, ctw, preferred_element_type=_F32,
                            precision=hi)

    rows = lax.broadcasted_iota(_I32, g.shape, 0)
    cols = lax.broadcasted_iota(_I32, g.shape, 1)
    diag = rows == cols
    tr = jnp.sum(jnp.where(diag, g, 0.0))
    total = jnp.sum(g)
    denom = 1.0 / float(n_c * n_c)
    l2_ref[...] = jnp.reshape((total - tr) * denom, (1, 1))
    nm_ref[...] = jnp.reshape(tr * denom, (1, 1))


def _preds_kernel(e_ref, c_ref, w_ref, bmat_ref, b_ref, orig_ref, y_ref):
    e = e_ref[...]
    b = b_ref[...]
    orig_ref[...] = jnp.dot(e, w_ref[...], preferred_element_type=_F32) + b
    a = jnp.dot(e, c_ref[...], preferred_element_type=_F32)
    y_ref[...] = jnp.dot(a, bmat_ref[...], preferred_element_type=_F32) + b


def _cross_topk_kernel(c_ref, x_ref, l1_ref, s_scr, cr_scr, st_scr,
                       *, n_total, k, tile, n_bisect):
    j = pl.program_id(0)
    n_steps = pl.num_programs(0)
    n_c = c_ref.shape[1]
    kf = float(k)
    inf = jnp.float32(jnp.inf)

    x = x_ref[...]
    cr = lax.dot_general(c_ref[...], x, (((0,), (0,)), ((), ())),
                         preferred_element_type=_F32)
    xsq = jnp.sum(x * x, axis=0, keepdims=True)
    s = xsq - 2.0 * cr
    col = lax.broadcasted_iota(_I32, s.shape, 1) + j * tile
    valid = col < n_total
    s_pad = jnp.where(valid, s, inf)
    s_scr[:, pl.ds(j * tile, tile)] = s_pad
    cr_scr[:, pl.ds(j * tile, tile)] = cr

    mn_c = jnp.min(s_pad, axis=1, keepdims=True)
    mx_c = jnp.max(jnp.where(valid, s, -inf), axis=1, keepdims=True)
    first = j == 0
    st_scr[:, 0:1] = jnp.where(first, mn_c, jnp.minimum(st_scr[:, 0:1], mn_c))
    st_scr[:, 1:2] = jnp.where(first, mx_c, jnp.maximum(st_scr[:, 1:2], mx_c))

    @pl.when(j == n_steps - 1)
    def _select():
        n_pad = n_steps * tile
        chunk = 6400
        starts = list(range(0, n_pad, chunk))
        lo0 = st_scr[:, 0:1] - 1.0
        hi0 = st_scr[:, 1:2]

        def bisect_body(_, carry):
            lo, hi = carry
            mid = 0.5 * lo + 0.5 * hi
            cnt = jnp.zeros((n_c, 1), dtype=_F32)
            for st in starts:
                sc = s_scr[:, pl.ds(st, chunk)]
                cnt = cnt + jnp.sum(jnp.where(sc <= mid, 1.0, 0.0),
                                    axis=1, keepdims=True)
            pred = cnt >= kf
            return jnp.where(pred, lo, mid), jnp.where(pred, mid, hi)

        lo, hi = lax.fori_loop(0, n_bisect, bisect_body, (lo0, hi0))

        n_lt = jnp.zeros((n_c, 1), dtype=_F32)
        sum_lt = jnp.zeros((n_c, 1), dtype=_F32)
        n_md = jnp.zeros((n_c, 1), dtype=_F32)
        sum_md = jnp.zeros((n_c, 1), dtype=_F32)
        for st in starts:
            sc = s_scr[:, pl.ds(st, chunk)]
            cc = cr_scr[:, pl.ds(st, chunk)]
            lt = sc <= lo
            md = jnp.logical_and(sc > lo, sc <= hi)
            n_lt = n_lt + jnp.sum(jnp.where(lt, 1.0, 0.0), axis=1,
                                  keepdims=True)
            sum_lt = sum_lt + jnp.sum(jnp.where(lt, cc, 0.0), axis=1,
                                      keepdims=True)
            n_md = n_md + jnp.sum(jnp.where(md, 1.0, 0.0), axis=1,
                                  keepdims=True)
            sum_md = sum_md + jnp.sum(jnp.where(md, cc, 0.0), axis=1,
                                      keepdims=True)

        m = kf - n_lt
        row_total = sum_lt + m * sum_md / jnp.maximum(n_md, 1.0)
        l1_ref[...] = jnp.reshape(jnp.sum(row_total) / (kf * float(n_c)),
                                  (1, 1))


def kernel(train_embedding, concept, train_embeddings_T, W_h, b_h, topk):
    bs, d = train_embedding.shape
    n_c = concept.shape[1]
    n = train_embeddings_T.shape[1]
    n_cls = W_h.shape[1]

    bmat, l2, nm = pl.pallas_call(
        _prep_kernel,
        out_shape=(
            jax.ShapeDtypeStruct((n_c, n_cls), _F32),
            jax.ShapeDtypeStruct((1, 1), _F32),
            jax.ShapeDtypeStruct((1, 1), _F32),
        ),
    )(concept, W_h)

    tile = 2048
    grid_m = bs // tile
    orig_pred, y_pred = pl.pallas_call(
        _preds_kernel,
        grid=(grid_m,),
        in_specs=[
            pl.BlockSpec((tile, d), lambda i: (i, 0)),
            pl.BlockSpec((d, n_c), lambda i: (0, 0)),
            pl.BlockSpec((d, n_cls), lambda i: (0, 0)),
            pl.BlockSpec((n_c, n_cls), lambda i: (0, 0)),
            pl.BlockSpec((1, n_cls), lambda i: (0, 0)),
        ],
        out_specs=(
            pl.BlockSpec((tile, n_cls), lambda i: (i, 0)),
            pl.BlockSpec((tile, n_cls), lambda i: (i, 0)),
        ),
        out_shape=(
            jax.ShapeDtypeStruct((bs, n_cls), _F32),
            jax.ShapeDtypeStruct((bs, n_cls), _F32),
        ),
        compiler_params=pltpu.CompilerParams(
            dimension_semantics=("parallel",)),
    )(train_embedding, concept, W_h, bmat, b_h.reshape(1, n_cls))

    ntile = 2560
    n_steps = (n + ntile - 1) // ntile
    n_pad = n_steps * ntile
    l1 = pl.pallas_call(
        functools.partial(_cross_topk_kernel, n_total=n, k=_K, tile=ntile,
                          n_bisect=24),
        grid=(n_steps,),
        in_specs=[
            pl.BlockSpec((d, n_c), lambda j: (0, 0)),
            pl.BlockSpec((d, ntile), lambda j: (0, j)),
        ],
        out_specs=pl.BlockSpec((1, 1), lambda j: (0, 0)),
        out_shape=jax.ShapeDtypeStruct((1, 1), _F32),
        scratch_shapes=[
            pltpu.VMEM((n_c, n_pad), _F32),
            pltpu.VMEM((n_c, n_pad), _F32),
            pltpu.VMEM((n_c, 128), _F32),
        ],
        compiler_params=pltpu.CompilerParams(
            dimension_semantics=("arbitrary",)),
    )(concept, train_embeddings_T)

    return (orig_pred, y_pred, l1[0, 0], l2[0, 0], nm[0, 0])


# speedup vs baseline: 1.0677x; 1.0246x over previous
import functools

import jax
import jax.numpy as jnp
from jax import lax
from jax.experimental import pallas as pl
from jax.experimental.pallas import tpu as pltpu

_K = 32

_F32 = jnp.float32
_I32 = jnp.int32


def _prep_kernel(c_ref, w_ref, bmat_ref, l2_ref, nm_ref):
    c = c_ref[...]
    hi = jax.lax.Precision.HIGHEST
    g = lax.dot_general(c, c, (((0,), (0,)), ((), ())),
                        preferred_element_type=_F32, precision=hi)
    n_c = g.shape[0]
    absg = jnp.abs(g)
    norm_inf = jnp.max(jnp.sum(absg, axis=1))
    norm_one = jnp.max(jnp.sum(absg, axis=0))
    x0 = g * (1.0 / (norm_inf * norm_one))

    def body(_, x):
        xg = jnp.dot(x, g, preferred_element_type=_F32, precision=hi)
        return 2.0 * x - jnp.dot(xg, x, preferred_element_type=_F32,
                                 precision=hi)

    ginv = lax.fori_loop(0, 24, body, x0)
    ctw = lax.dot_general(c, w_ref[...], (((0,), (0,)), ((), ())),
                          preferred_element_type=_F32, precision=hi)
    bmat_ref[...] = jnp.dot(ginv, ctw, preferred_element_type=_F32,
                            precision=hi)

    rows = lax.broadcasted_iota(_I32, g.shape, 0)
    cols = lax.broadcasted_iota(_I32, g.shape, 1)
    diag = rows == cols
    tr = jnp.sum(jnp.where(diag, g, 0.0))
    total = jnp.sum(g)
    denom = 1.0 / float(n_c * n_c)
    l2_ref[...] = jnp.reshape((total - tr) * denom, (1, 1))
    nm_ref[...] = jnp.reshape(tr * denom, (1, 1))


def _preds_kernel(e_ref, c_ref, w_ref, bmat_ref, b_ref, orig_ref, y_ref):
    e = e_ref[...]
    b = b_ref[...]
    orig_ref[...] = jnp.dot(e, w_ref[...], preferred_element_type=_F32) + b
    a = jnp.dot(e, c_ref[...], preferred_element_type=_F32)
    y_ref[...] = jnp.dot(a, bmat_ref[...], preferred_element_type=_F32) + b


def _cross_topk_kernel(c_ref, x_ref, l1_ref, s_scr, cr_scr, st_scr,
                       *, n_total, k, tile, n_bisect):
    j = pl.program_id(0)
    n_steps = pl.num_programs(0)
    n_c = c_ref.shape[1]
    kf = float(k)
    inf = jnp.float32(jnp.inf)

    x = x_ref[...]
    cr = lax.dot_general(c_ref[...], x, (((0,), (0,)), ((), ())),
                         preferred_element_type=_F32)
    xsq = jnp.sum(x * x, axis=0, keepdims=True)
    s = xsq - 2.0 * cr
    col = lax.broadcasted_iota(_I32, s.shape, 1) + j * tile
    valid = col < n_total
    s_pad = jnp.where(valid, s, inf)
    s_scr[:, pl.ds(j * tile, tile)] = s_pad
    cr_scr[:, pl.ds(j * tile, tile)] = cr

    mn_c = jnp.min(s_pad, axis=1, keepdims=True)
    mx_c = jnp.max(jnp.where(valid, s, -inf), axis=1, keepdims=True)
    first = j == 0
    st_scr[:, 0:1] = jnp.where(first, mn_c, jnp.minimum(st_scr[:, 0:1], mn_c))
    st_scr[:, 1:2] = jnp.where(first, mx_c, jnp.maximum(st_scr[:, 1:2], mx_c))

    @pl.when(j == n_steps - 1)
    def _select():
        n_pad = n_steps * tile
        chunk = 6400
        starts = list(range(0, n_pad, chunk))
        lo0 = st_scr[:, 0:1] - 1.0
        hi0 = st_scr[:, 1:2]

        def bisect_body(_, carry):
            lo, hi = carry
            mid = 0.5 * lo + 0.5 * hi
            cnt = jnp.zeros((n_c, 1), dtype=_F32)
            for st in starts:
                sc = s_scr[:, pl.ds(st, chunk)]
                cnt = cnt + jnp.sum(jnp.where(sc <= mid, 1.0, 0.0),
                                    axis=1, keepdims=True)
            pred = cnt >= kf
            return jnp.where(pred, lo, mid), jnp.where(pred, mid, hi)

        lo, hi = lax.fori_loop(0, n_bisect, bisect_body, (lo0, hi0))

        n_lt = jnp.zeros((n_c, 1), dtype=_F32)
        sum_lt = jnp.zeros((n_c, 1), dtype=_F32)
        n_md = jnp.zeros((n_c, 1), dtype=_F32)
        sum_md = jnp.zeros((n_c, 1), dtype=_F32)
        for st in starts:
            sc = s_scr[:, pl.ds(st, chunk)]
            cc = cr_scr[:, pl.ds(st, chunk)]
            lt = sc <= lo
            md = jnp.logical_and(sc > lo, sc <= hi)
            n_lt = n_lt + jnp.sum(jnp.where(lt, 1.0, 0.0), axis=1,
                                  keepdims=True)
            sum_lt = sum_lt + jnp.sum(jnp.where(lt, cc, 0.0), axis=1,
                                      keepdims=True)
            n_md = n_md + jnp.sum(jnp.where(md, 1.0, 0.0), axis=1,
                                  keepdims=True)
            sum_md = sum_md + jnp.sum(jnp.where(md, cc, 0.0), axis=1,
                                      keepdims=True)

        m = kf - n_lt
        row_total = sum_lt + m * sum_md / jnp.maximum(n_md, 1.0)
        l1_ref[...] = jnp.reshape(jnp.sum(row_total) / (kf * float(n_c)),
                                  (1, 1))


def kernel(train_embedding, concept, train_embeddings_T, W_h, b_h, topk):
    bs, d = train_embedding.shape
    n_c = concept.shape[1]
    n = train_embeddings_T.shape[1]
    n_cls = W_h.shape[1]

    bmat, l2, nm = pl.pallas_call(
        _prep_kernel,
        out_shape=(
            jax.ShapeDtypeStruct((n_c, n_cls), _F32),
            jax.ShapeDtypeStruct((1, 1), _F32),
            jax.ShapeDtypeStruct((1, 1), _F32),
        ),
    )(concept, W_h)

    tile = 2048
    grid_m = bs // tile
    orig_pred, y_pred = pl.pallas_call(
        _preds_kernel,
        grid=(grid_m,),
        in_specs=[
            pl.BlockSpec((tile, d), lambda i: (i, 0)),
            pl.BlockSpec((d, n_c), lambda i: (0, 0)),
            pl.BlockSpec((d, n_cls), lambda i: (0, 0)),
            pl.BlockSpec((n_c, n_cls), lambda i: (0, 0)),
            pl.BlockSpec((1, n_cls), lambda i: (0, 0)),
        ],
        out_specs=(
            pl.BlockSpec((tile, n_cls), lambda i: (i, 0)),
            pl.BlockSpec((tile, n_cls), lambda i: (i, 0)),
        ),
        out_shape=(
            jax.ShapeDtypeStruct((bs, n_cls), _F32),
            jax.ShapeDtypeStruct((bs, n_cls), _F32),
        ),
        compiler_params=pltpu.CompilerParams(
            dimension_semantics=("parallel",)),
    )(train_embedding, concept, W_h, bmat, b_h.reshape(1, n_cls))

    ntile = 2560
    n_steps = (n + ntile - 1) // ntile
    n_pad = n_steps * ntile
    l1 = pl.pallas_call(
        functools.partial(_cross_topk_kernel, n_total=n, k=_K, tile=ntile,
                          n_bisect=18),
        grid=(n_steps,),
        in_specs=[
            pl.BlockSpec((d, n_c), lambda j: (0, 0)),
            pl.BlockSpec((d, ntile), lambda j: (0, j)),
        ],
        out_specs=pl.BlockSpec((1, 1), lambda j: (0, 0)),
        out_shape=jax.ShapeDtypeStruct((1, 1), _F32),
        scratch_shapes=[
            pltpu.VMEM((n_c, n_pad), _F32),
            pltpu.VMEM((n_c, n_pad), _F32),
            pltpu.VMEM((n_c, 128), _F32),
        ],
        compiler_params=pltpu.CompilerParams(
            dimension_semantics=("arbitrary",)),
    )(concept, train_embeddings_T)

    return (orig_pred, y_pred, l1[0, 0], l2[0, 0], nm[0, 0])


# speedup vs baseline: 1.0738x; 1.0057x over previous
import functools

import jax
import jax.numpy as jnp
from jax import lax
from jax.experimental import pallas as pl
from jax.experimental.pallas import tpu as pltpu

_K = 32

_F32 = jnp.float32
_I32 = jnp.int32


def _prep(c_ref, w_ref, bmat_scr, l2_ref, nm_ref):
    c = c_ref[...]
    hi = jax.lax.Precision.HIGHEST
    g = lax.dot_general(c, c, (((0,), (0,)), ((), ())),
                        preferred_element_type=_F32, precision=hi)
    n_c = g.shape[0]
    absg = jnp.abs(g)
    norm_inf = jnp.max(jnp.sum(absg, axis=1))
    norm_one = jnp.max(jnp.sum(absg, axis=0))
    x0 = g * (1.0 / (norm_inf * norm_one))

    def body(_, x):
        xg = jnp.dot(x, g, preferred_element_type=_F32, precision=hi)
        return 2.0 * x - jnp.dot(xg, x, preferred_element_type=_F32,
                                 precision=hi)

    ginv = lax.fori_loop(0, 24, body, x0)
    ctw = lax.dot_general(c, w_ref[...], (((0,), (0,)), ((), ())),
                          preferred_element_type=_F32, precision=hi)
    bmat_scr[...] = jnp.dot(ginv, ctw, preferred_element_type=_F32,
                            precision=hi)

    rows = lax.broadcasted_iota(_I32, g.shape, 0)
    cols = lax.broadcasted_iota(_I32, g.shape, 1)
    diag = rows == cols
    tr = jnp.sum(jnp.where(diag, g, 0.0))
    total = jnp.sum(g)
    denom = 1.0 / float(n_c * n_c)
    l2_ref[...] = jnp.reshape((total - tr) * denom, (1, 1))
    nm_ref[...] = jnp.reshape(tr * denom, (1, 1))


def _preds_kernel(e_ref, c_ref, w_ref, b_ref,
                  orig_ref, y_ref, l2_ref, nm_ref, bmat_scr):
    @pl.when(pl.program_id(0) == 0)
    def _p():
        _prep(c_ref, w_ref, bmat_scr, l2_ref, nm_ref)

    e = e_ref[...]
    b = b_ref[...]
    orig_ref[...] = jnp.dot(e, w_ref[...], preferred_element_type=_F32) + b
    a = jnp.dot(e, c_ref[...], preferred_element_type=_F32)
    y_ref[...] = jnp.dot(a, bmat_scr[...], preferred_element_type=_F32) + b


def _cross_topk_kernel(c_ref, x_ref, l1_ref, s_scr, cr_scr, st_scr,
                       *, n_total, k, tile, n_bisect):
    j = pl.program_id(0)
    n_steps = pl.num_programs(0)
    n_c = c_ref.shape[1]
    kf = float(k)
    inf = jnp.float32(jnp.inf)

    x = x_ref[...]
    cr = lax.dot_general(c_ref[...], x, (((0,), (0,)), ((), ())),
                         preferred_element_type=_F32)
    xsq = jnp.sum(x * x, axis=0, keepdims=True)
    s = xsq - 2.0 * cr
    col = lax.broadcasted_iota(_I32, s.shape, 1) + j * tile
    valid = col < n_total
    s_pad = jnp.where(valid, s, inf)
    s_scr[:, pl.ds(j * tile, tile)] = s_pad
    cr_scr[:, pl.ds(j * tile, tile)] = cr

    mn_c = jnp.min(s_pad, axis=1, keepdims=True)
    mx_c = jnp.max(jnp.where(valid, s, -inf), axis=1, keepdims=True)
    first = j == 0
    st_scr[:, 0:1] = jnp.where(first, mn_c, jnp.minimum(st_scr[:, 0:1], mn_c))
    st_scr[:, 1:2] = jnp.where(first, mx_c, jnp.maximum(st_scr[:, 1:2], mx_c))

    @pl.when(j == n_steps - 1)
    def _select():
        n_pad = n_steps * tile
        chunk = 6400
        starts = list(range(0, n_pad, chunk))
        lo0 = st_scr[:, 0:1] - 1.0
        hi0 = st_scr[:, 1:2]

        def bisect_body(_, carry):
            lo, hi = carry
            mid = 0.5 * lo + 0.5 * hi
            cnt = jnp.zeros((n_c, 1), dtype=_F32)
            for st in starts:
                sc = s_scr[:, pl.ds(st, chunk)]
                cnt = cnt + jnp.sum(jnp.where(sc <= mid, 1.0, 0.0),
                                    axis=1, keepdims=True)
            pred = cnt >= kf
            return jnp.where(pred, lo, mid), jnp.where(pred, mid, hi)

        lo, hi = lax.fori_loop(0, n_bisect, bisect_body, (lo0, hi0))

        n_lt = jnp.zeros((n_c, 1), dtype=_F32)
        sum_lt = jnp.zeros((n_c, 1), dtype=_F32)
        n_md = jnp.zeros((n_c, 1), dtype=_F32)
        sum_md = jnp.zeros((n_c, 1), dtype=_F32)
        for st in starts:
            sc = s_scr[:, pl.ds(st, chunk)]
            cc = cr_scr[:, pl.ds(st, chunk)]
            lt = sc <= lo
            md = jnp.logical_and(sc > lo, sc <= hi)
            n_lt = n_lt + jnp.sum(jnp.where(lt, 1.0, 0.0), axis=1,
                                  keepdims=True)
            sum_lt = sum_lt + jnp.sum(jnp.where(lt, cc, 0.0), axis=1,
                                      keepdims=True)
            n_md = n_md + jnp.sum(jnp.where(md, 1.0, 0.0), axis=1,
                                  keepdims=True)
            sum_md = sum_md + jnp.sum(jnp.where(md, cc, 0.0), axis=1,
                                      keepdims=True)

        m = kf - n_lt
        row_total = sum_lt + m * sum_md / jnp.maximum(n_md, 1.0)
        l1_ref[...] = jnp.reshape(jnp.sum(row_total) / (kf * float(n_c)),
                                  (1, 1))


def kernel(train_embedding, concept, train_embeddings_T, W_h, b_h, topk):
    bs, d = train_embedding.shape
    n_c = concept.shape[1]
    n = train_embeddings_T.shape[1]
    n_cls = W_h.shape[1]

    tile = 2048
    grid_m = bs // tile
    orig_pred, y_pred, l2, nm = pl.pallas_call(
        _preds_kernel,
        grid=(grid_m,),
        in_specs=[
            pl.BlockSpec((tile, d), lambda i: (i, 0)),
            pl.BlockSpec((d, n_c), lambda i: (0, 0)),
            pl.BlockSpec((d, n_cls), lambda i: (0, 0)),
            pl.BlockSpec((1, n_cls), lambda i: (0, 0)),
        ],
        out_specs=(
            pl.BlockSpec((tile, n_cls), lambda i: (i, 0)),
            pl.BlockSpec((tile, n_cls), lambda i: (i, 0)),
            pl.BlockSpec((1, 1), lambda i: (0, 0)),
            pl.BlockSpec((1, 1), lambda i: (0, 0)),
        ),
        out_shape=(
            jax.ShapeDtypeStruct((bs, n_cls), _F32),
            jax.ShapeDtypeStruct((bs, n_cls), _F32),
            jax.ShapeDtypeStruct((1, 1), _F32),
            jax.ShapeDtypeStruct((1, 1), _F32),
        ),
        scratch_shapes=[pltpu.VMEM((n_c, n_cls), _F32)],
        compiler_params=pltpu.CompilerParams(
            dimension_semantics=("arbitrary",)),
    )(train_embedding, concept, W_h, b_h.reshape(1, n_cls))

    ntile = 2560
    n_steps = (n + ntile - 1) // ntile
    n_pad = n_steps * ntile
    l1 = pl.pallas_call(
        functools.partial(_cross_topk_kernel, n_total=n, k=_K, tile=ntile,
                          n_bisect=18),
        grid=(n_steps,),
        in_specs=[
            pl.BlockSpec((d, n_c), lambda j: (0, 0)),
            pl.BlockSpec((d, ntile), lambda j: (0, j)),
        ],
        out_specs=pl.BlockSpec((1, 1), lambda j: (0, 0)),
        out_shape=jax.ShapeDtypeStruct((1, 1), _F32),
        scratch_shapes=[
            pltpu.VMEM((n_c, n_pad), _F32),
            pltpu.VMEM((n_c, n_pad), _F32),
            pltpu.VMEM((n_c, 128), _F32),
        ],
        compiler_params=pltpu.CompilerParams(
            dimension_semantics=("arbitrary",)),
    )(concept, train_embeddings_T)

    return (orig_pred, y_pred, l1[0, 0], l2[0, 0], nm[0, 0])


# speedup vs baseline: 1.0861x; 1.0114x over previous
import functools

import jax
import jax.numpy as jnp
from jax import lax
from jax.experimental import pallas as pl
from jax.experimental.pallas import tpu as pltpu

_K = 32

_F32 = jnp.float32
_I32 = jnp.int32


def _prep(c_ref, w_ref, bmat_scr, l2_ref, nm_ref):
    c = c_ref[...]
    hi = jax.lax.Precision.HIGHEST
    g = lax.dot_general(c, c, (((0,), (0,)), ((), ())),
                        preferred_element_type=_F32, precision=hi)
    n_c = g.shape[0]
    absg = jnp.abs(g)
    norm_inf = jnp.max(jnp.sum(absg, axis=1))
    norm_one = jnp.max(jnp.sum(absg, axis=0))
    x0 = g * (1.0 / (norm_inf * norm_one))

    def body(_, x):
        xg = jnp.dot(x, g, preferred_element_type=_F32, precision=hi)
        return 2.0 * x - jnp.dot(xg, x, preferred_element_type=_F32,
                                 precision=hi)

    ginv = lax.fori_loop(0, 24, body, x0)
    ctw = lax.dot_general(c, w_ref[...], (((0,), (0,)), ((), ())),
                          preferred_element_type=_F32, precision=hi)
    bmat_scr[...] = jnp.dot(ginv, ctw, preferred_element_type=_F32,
                            precision=hi)

    rows = lax.broadcasted_iota(_I32, g.shape, 0)
    cols = lax.broadcasted_iota(_I32, g.shape, 1)
    diag = rows == cols
    tr = jnp.sum(jnp.where(diag, g, 0.0))
    total = jnp.sum(g)
    denom = 1.0 / float(n_c * n_c)
    l2_ref[...] = jnp.reshape((total - tr) * denom, (1, 1))
    nm_ref[...] = jnp.reshape(tr * denom, (1, 1))


def _preds_kernel(e_ref, c_ref, w_ref, b_ref,
                  orig_ref, y_ref, l2_ref, nm_ref, bmat_scr):
    @pl.when(pl.program_id(0) == 0)
    def _p():
        _prep(c_ref, w_ref, bmat_scr, l2_ref, nm_ref)

    e = e_ref[...]
    b = b_ref[...]
    orig_ref[...] = jnp.dot(e, w_ref[...], preferred_element_type=_F32) + b
    a = jnp.dot(e, c_ref[...], preferred_element_type=_F32)
    y_ref[...] = jnp.dot(a, bmat_scr[...], preferred_element_type=_F32) + b


def _cross_topk_kernel(c_ref, x_ref, l1_ref, s_scr, cr_scr, st_scr,
                       *, n_total, k, tile, n_bisect):
    j = pl.program_id(0)
    n_steps = pl.num_programs(0)
    n_c = c_ref.shape[1]
    kf = float(k)
    inf = jnp.float32(jnp.inf)

    x = x_ref[...]
    cr = lax.dot_general(c_ref[...], x, (((0,), (0,)), ((), ())),
                         preferred_element_type=_F32)
    xsq = jnp.sum(x * x, axis=0, keepdims=True)
    s = xsq - 2.0 * cr
    col = lax.broadcasted_iota(_I32, s.shape, 1) + j * tile
    valid = col < n_total
    s_pad = jnp.where(valid, s, inf)
    s_scr[:, pl.ds(j * tile, tile)] = s_pad
    cr_scr[:, pl.ds(j * tile, tile)] = cr

    mn_c = jnp.min(s_pad, axis=1, keepdims=True)
    mx_c = jnp.max(jnp.where(valid, s, -inf), axis=1, keepdims=True)
    first = j == 0
    st_scr[:, 0:1] = jnp.where(first, mn_c, jnp.minimum(st_scr[:, 0:1], mn_c))
    st_scr[:, 1:2] = jnp.where(first, mx_c, jnp.maximum(st_scr[:, 1:2], mx_c))

    @pl.when(j == n_steps - 1)
    def _select():
        n_pad = n_steps * tile
        chunk = 6400
        starts = list(range(0, n_pad, chunk))
        lo0 = st_scr[:, 0:1] - 1.0
        hi0 = st_scr[:, 1:2]

        def bisect_body(_, carry):
            lo, hi = carry
            mid = 0.5 * lo + 0.5 * hi
            cnt = jnp.zeros((n_c, 1), dtype=_F32)
            for st in starts:
                sc = s_scr[:, pl.ds(st, chunk)]
                cnt = cnt + jnp.sum(jnp.where(sc <= mid, 1.0, 0.0),
                                    axis=1, keepdims=True)
            pred = cnt >= kf
            return jnp.where(pred, lo, mid), jnp.where(pred, mid, hi)

        lo, hi = lax.fori_loop(0, n_bisect, bisect_body, (lo0, hi0))

        n_lt = jnp.zeros((n_c, 1), dtype=_F32)
        sum_lt = jnp.zeros((n_c, 1), dtype=_F32)
        n_md = jnp.zeros((n_c, 1), dtype=_F32)
        sum_md = jnp.zeros((n_c, 1), dtype=_F32)
        for st in starts:
            sc = s_scr[:, pl.ds(st, chunk)]
            cc = cr_scr[:, pl.ds(st, chunk)]
            lt = sc <= lo
            md = jnp.logical_and(sc > lo, sc <= hi)
            n_lt = n_lt + jnp.sum(jnp.where(lt, 1.0, 0.0), axis=1,
                                  keepdims=True)
            sum_lt = sum_lt + jnp.sum(jnp.where(lt, cc, 0.0), axis=1,
                                      keepdims=True)
            n_md = n_md + jnp.sum(jnp.where(md, 1.0, 0.0), axis=1,
                                  keepdims=True)
            sum_md = sum_md + jnp.sum(jnp.where(md, cc, 0.0), axis=1,
                                      keepdims=True)

        m = kf - n_lt
        row_total = sum_lt + m * sum_md / jnp.maximum(n_md, 1.0)
        l1_ref[...] = jnp.reshape(jnp.sum(row_total) / (kf * float(n_c)),
                                  (1, 1))


def kernel(train_embedding, concept, train_embeddings_T, W_h, b_h, topk):
    bs, d = train_embedding.shape
    n_c = concept.shape[1]
    n = train_embeddings_T.shape[1]
    n_cls = W_h.shape[1]

    tile = 2048
    grid_m = bs // tile
    orig_pred, y_pred, l2, nm = pl.pallas_call(
        _preds_kernel,
        grid=(grid_m,),
        in_specs=[
            pl.BlockSpec((tile, d), lambda i: (i, 0)),
            pl.BlockSpec((d, n_c), lambda i: (0, 0)),
            pl.BlockSpec((d, n_cls), lambda i: (0, 0)),
            pl.BlockSpec((1, n_cls), lambda i: (0, 0)),
        ],
        out_specs=(
            pl.BlockSpec((tile, n_cls), lambda i: (i, 0)),
            pl.BlockSpec((tile, n_cls), lambda i: (i, 0)),
            pl.BlockSpec((1, 1), lambda i: (0, 0)),
            pl.BlockSpec((1, 1), lambda i: (0, 0)),
        ),
        out_shape=(
            jax.ShapeDtypeStruct((bs, n_cls), _F32),
            jax.ShapeDtypeStruct((bs, n_cls), _F32),
            jax.ShapeDtypeStruct((1, 1), _F32),
            jax.ShapeDtypeStruct((1, 1), _F32),
        ),
        scratch_shapes=[pltpu.VMEM((n_c, n_cls), _F32)],
        compiler_params=pltpu.CompilerParams(
            dimension_semantics=("arbitrary",)),
    )(train_embedding, concept, W_h, b_h.reshape(1, n_cls))

    ntile = 5120
    n_steps = (n + ntile - 1) // ntile
    n_pad = n_steps * ntile
    l1 = pl.pallas_call(
        functools.partial(_cross_topk_kernel, n_total=n, k=_K, tile=ntile,
                          n_bisect=18),
        grid=(n_steps,),
        in_specs=[
            pl.BlockSpec((d, n_c), lambda j: (0, 0)),
            pl.BlockSpec((d, ntile), lambda j: (0, j)),
        ],
        out_specs=pl.BlockSpec((1, 1), lambda j: (0, 0)),
        out_shape=jax.ShapeDtypeStruct((1, 1), _F32),
        scratch_shapes=[
            pltpu.VMEM((n_c, n_pad), _F32),
            pltpu.VMEM((n_c, n_pad), _F32),
            pltpu.VMEM((n_c, 128), _F32),
        ],
        compiler_params=pltpu.CompilerParams(
            dimension_semantics=("arbitrary",)),
    )(concept, train_embeddings_T)

    return (orig_pred, y_pred, l1[0, 0], l2[0, 0], nm[0, 0])
